# Initial kernel scaffold; baseline (speedup 1.0000x reference)
import functools

import jax
import jax.numpy as jnp
from jax import lax
from jax.experimental import pallas as pl
from jax.experimental.pallas import tpu as pltpu
from jax.experimental.pallas import tpu_sc as plsc

F32 = jnp.float32

_NC = 2
_NS = 16
_L = 16

_DEG_CHUNK = 100
_AGG_CHUNK = 80


def _deg_body(dst_hbm, deg_out, dstbuf, onesbuf, zbuf, acc):
    c = lax.axis_index("c")
    s = lax.axis_index("s")
    w = c * _NS + s
    rows = dstbuf.shape[0]
    n = acc.shape[0]
    rpt = n // _NS

    def fill_ones(i, _):
        onesbuf[i] = jnp.ones((_L,), F32)
        return 0

    lax.fori_loop(0, onesbuf.shape[0], fill_ones, 0)

    def fill_zero(i, _):
        zbuf[i] = jnp.zeros((_L,), F32)
        return 0

    lax.fori_loop(0, rpt, fill_zero, 0)
    pltpu.sync_copy(zbuf, acc.at[pl.ds(s * rpt, rpt)])

    pltpu.sync_copy(dst_hbm.at[pl.ds(w * rows, rows)], dstbuf)
    plsc.subcore_barrier()

    def hist(j, _):
        pltpu.sync_copy(onesbuf, acc.at[dstbuf.at[j]], add=True)
        return 0

    lax.fori_loop(0, rows, hist, 0)
    plsc.subcore_barrier()
    pltpu.sync_copy(acc.at[pl.ds(s * rpt, rpt)],
                    deg_out.at[c, pl.ds(s * rpt, rpt), :])


def _deg_call(dstd, n):
    mesh = plsc.VectorSubcoreMesh(core_axis_name="c", subcore_axis_name="s")
    rows = dstd.shape[0] // (_NC * _NS)
    fn = pl.kernel(
        _deg_body,
        out_type=jax.ShapeDtypeStruct((_NC, n, _L), F32),
        mesh=mesh,
        scratch_types=[
            pltpu.VMEM((rows, _DEG_CHUNK), jnp.int32),
            pltpu.VMEM((_DEG_CHUNK, _L), F32),
            pltpu.VMEM((n // _NS, _L), F32),
            pltpu.VMEM_SHARED((n, _L), F32),
        ],
    )
    return fn(dstd)


def _agg_body(yv_hbm, srcv_hbm, dstv_hbm, agg_out,
              srcbuf, dstbuf, rb0, rb1, zbuf, acc, sem0, sem1):
    c = lax.axis_index("c")
    s = lax.axis_index("s")
    rows = srcbuf.shape[0]
    n = acc.shape[0]
    rpt = n // _NS
    zrows = zbuf.shape[0]

    def fill_zero(i, _):
        for k in range(_AGG_CHUNK * 4 // _L):
            zbuf[i, pl.ds(k * _L, _L)] = jnp.zeros((_L,), F32)
        return 0

    lax.fori_loop(0, zrows, fill_zero, 0)
    for k in range(rpt // zrows):
        pltpu.sync_copy(zbuf, acc.at[pl.ds(s * rpt + k * zrows, zrows)])

    pltpu.sync_copy(srcv_hbm.at[pl.ds(s * rows, rows)], srcbuf)
    pltpu.sync_copy(dstv_hbm.at[pl.ds(s * rows, rows)], dstbuf)

    def remap(i, _):
        for k in range(_AGG_CHUNK // _L):
            v = srcbuf[i, pl.ds(k * _L, _L)]
            srcbuf[i, pl.ds(k * _L, _L)] = v * 2 + c
        return 0

    lax.fori_loop(0, rows, remap, 0)
    plsc.subcore_barrier()

    def gather(j, rb, sem):
        pltpu.async_copy(yv_hbm.at[srcbuf.at[j]], rb, sem)

    def gwait(rb, sem):
        pltpu.make_async_copy(yv_hbm.at[srcbuf.at[0]], rb, sem).wait()

    def scat(j, rb):
        pltpu.sync_copy(rb, acc.at[dstbuf.at[j]], add=True)

    gather(0, rb0, sem0)

    def pipe(i, _):
        j0 = i * 2
        gwait(rb0, sem0)
        gather(j0 + 1, rb1, sem1)
        scat(j0, rb0)
        gwait(rb1, sem1)
        gather(j0 + 2, rb0, sem0)
        scat(j0 + 1, rb1)
        return 0

    lax.fori_loop(0, (rows - 1) // 2, pipe, 0)
    gwait(rb0, sem0)
    scat(rows - 1, rb0)

    plsc.subcore_barrier()
    pltpu.sync_copy(acc.at[pl.ds(s * rpt, rpt)],
                    agg_out.at[c, pl.ds(s * rpt, rpt), :])


def _agg_call(yv, srcv, dstv, n):
    mesh = plsc.VectorSubcoreMesh(core_axis_name="c", subcore_axis_name="s")
    rows = srcv.shape[0] // _NS
    fn = pl.kernel(
        _agg_body,
        out_type=jax.ShapeDtypeStruct((_NC, n, 128), F32),
        mesh=mesh,
        scratch_types=[
            pltpu.VMEM((rows, _AGG_CHUNK), jnp.int32),
            pltpu.VMEM((rows, _AGG_CHUNK), jnp.int32),
            pltpu.VMEM((_AGG_CHUNK, 128), F32),
            pltpu.VMEM((_AGG_CHUNK, 128), F32),
            pltpu.VMEM((125, 128), F32),
            pltpu.VMEM_SHARED((n, 128), F32),
            pltpu.SemaphoreType.DMA,
            pltpu.SemaphoreType.DMA,
        ],
    )
    return fn(yv, srcv, dstv)


def _mm_scale_body(x_ref, w_ref, deg_ref, o_ref):
    deg = deg_ref[0][:, :1] + deg_ref[1][:, :1] + 1.0
    dinv = lax.rsqrt(deg)
    xw = jnp.dot(x_ref[...], w_ref[...], preferred_element_type=F32)
    o_ref[...] = xw * dinv


def _mm_scale(x, w, deg, blk):
    n, d_in = x.shape
    d_out = w.shape[1]
    grid = n // blk
    return pl.pallas_call(
        _mm_scale_body,
        grid=(grid,),
        in_specs=[
            pl.BlockSpec((blk, d_in), lambda i: (i, 0)),
            pl.BlockSpec((d_in, d_out), lambda i: (0, 0)),
            pl.BlockSpec((_NC, blk, _L), lambda i: (0, i, 0)),
        ],
        out_specs=pl.BlockSpec((blk, d_out), lambda i: (i, 0)),
        out_shape=jax.ShapeDtypeStruct((n, d_out), F32),
    )(x, w, deg)


def _layer_body(agg_ref, y_ref, deg_ref, b_ref, w_ref, o_ref):
    deg = deg_ref[0][:, :1] + deg_ref[1][:, :1] + 1.0
    dinv = lax.rsqrt(deg)
    aggf = jnp.concatenate([agg_ref[0], agg_ref[1]], axis=-1) + y_ref[...]
    h = jnp.maximum(aggf * dinv + b_ref[...], 0.0)
    o_ref[...] = jnp.dot(h, w_ref[...], preferred_element_type=F32) * dinv


def _layer(agg, y, deg, b, w, blk):
    n, d = y.shape
    grid = n // blk
    return pl.pallas_call(
        _layer_body,
        grid=(grid,),
        in_specs=[
            pl.BlockSpec((_NC, blk, 128), lambda i: (0, i, 0)),
            pl.BlockSpec((blk, d), lambda i: (i, 0)),
            pl.BlockSpec((_NC, blk, _L), lambda i: (0, i, 0)),
            pl.BlockSpec((1, d), lambda i: (0, 0)),
            pl.BlockSpec((d, d), lambda i: (0, 0)),
        ],
        out_specs=pl.BlockSpec((blk, d), lambda i: (i, 0)),
        out_shape=jax.ShapeDtypeStruct((n, d), F32),
    )(agg, y, deg, b, w)


def _head_body(agg_ref, y_ref, deg_ref, b_ref, batch_ref, wfc_ref, bfc_ref,
               o_ref, pool_ref, *, nblk, g):
    i = pl.program_id(0)
    deg = deg_ref[0][:, :1] + deg_ref[1][:, :1] + 1.0
    dinv = lax.rsqrt(deg)
    aggf = jnp.concatenate([agg_ref[0], agg_ref[1]], axis=-1) + y_ref[...]
    h = jnp.maximum(aggf * dinv + b_ref[...], 0.0)
    bvec = batch_ref[0, 0, :]
    oh = (bvec[:, None] ==
          lax.broadcasted_iota(jnp.int32, (bvec.shape[0], g), 1)).astype(F32)
    seg = lax.dot_general(oh, h, (((0,), (0,)), ((), ())),
                          preferred_element_type=F32)

    @pl.when(i == 0)
    def _():
        pool_ref[...] = jnp.zeros_like(pool_ref)

    pool_ref[...] += seg

    @pl.when(i == nblk - 1)
    def _():
        o_ref[...] = (jnp.dot(pool_ref[...], wfc_ref[...],
                              preferred_element_type=F32) + bfc_ref[...])


def _head(agg, y, deg, b, batchv, wfc, bfc, blk, g):
    n, d = y.shape
    d_out = wfc.shape[1]
    grid = n // blk
    body = functools.partial(_head_body, nblk=grid, g=g)
    return pl.pallas_call(
        body,
        grid=(grid,),
        in_specs=[
            pl.BlockSpec((_NC, blk, 128), lambda i: (0, i, 0)),
            pl.BlockSpec((blk, d), lambda i: (i, 0)),
            pl.BlockSpec((_NC, blk, _L), lambda i: (0, i, 0)),
            pl.BlockSpec((1, d), lambda i: (0, 0)),
            pl.BlockSpec((1, 1, blk), lambda i: (i, 0, 0)),
            pl.BlockSpec((d, d_out), lambda i: (0, 0)),
            pl.BlockSpec((1, d_out), lambda i: (0, 0)),
        ],
        out_specs=pl.BlockSpec((g, d_out), lambda i: (0, 0)),
        out_shape=jax.ShapeDtypeStruct((g, d_out), F32),
        scratch_shapes=[pltpu.VMEM((g, d), F32)],
    )(agg, y, deg, b, batchv, wfc, bfc)


def kernel(x, edge_index, batch, W1, b1, W2, b2, Wfc, bfc):
    n, d_in = x.shape
    e = edge_index.shape[1]
    d_h = W1.shape[1]
    g = 64
    blk = 1000
    assert d_h == 256 and n % (blk * 2) == 0
    assert e % (_DEG_CHUNK * _NC * _NS) == 0 and e % (_AGG_CHUNK * _NS) == 0

    src = edge_index[0]
    dst = edge_index[1]
    dstd = dst.reshape(e // _DEG_CHUNK, _DEG_CHUNK)
    srcv = src.reshape(e // _AGG_CHUNK, _AGG_CHUNK)
    dstv = dst.reshape(e // _AGG_CHUNK, _AGG_CHUNK)
    batchv = batch.reshape(n // blk, 1, blk)

    deg = _deg_call(dstd, n)
    y1 = _mm_scale(x, W1, deg, blk)
    agg1 = _agg_call(y1.reshape(2 * n, 128), srcv, dstv, n)
    y2 = _layer(agg1, y1, deg, b1.reshape(1, -1), W2, blk)
    agg2 = _agg_call(y2.reshape(2 * n, 128), srcv, dstv, n)
    return _head(agg2, y2, deg, b2.reshape(1, -1), batchv, Wfc,
                 bfc.reshape(1, -1), blk, g)


# speedup vs baseline: 6.5315x; 6.5315x over previous
import functools

import jax
import jax.numpy as jnp
from jax import lax
from jax.experimental import pallas as pl
from jax.experimental.pallas import tpu as pltpu
from jax.experimental.pallas import tpu_sc as plsc

F32 = jnp.float32

_NC = 2
_NS = 16
_L = 16

_DEG_CHUNK = 125
_AGG_CHUNK = 64


def _round_up(v, m):
    return (v + m - 1) // m * m


def _deg_body(dst_hbm, deg_out, dstbuf, onesbuf, zbuf, acc):
    c = lax.axis_index("c")
    s = lax.axis_index("s")
    w = c * _NS + s
    rows = dstbuf.shape[0]
    rpt = acc.shape[0] // _NS
    zrows = zbuf.shape[0]

    def fill_ones(i, _):
        for k in range(128 // _L):
            onesbuf[i, pl.ds(k * _L, _L)] = jnp.ones((_L,), F32)
        return 0

    lax.fori_loop(0, onesbuf.shape[0], fill_ones, 0)

    def fill_zero(i, _):
        for k in range(128 // _L):
            zbuf[i, pl.ds(k * _L, _L)] = jnp.zeros((_L,), F32)
        return 0

    lax.fori_loop(0, zrows, fill_zero, 0)
    for off in range(0, rpt - zrows + 1, zrows):
        pltpu.sync_copy(zbuf, acc.at[pl.ds(s * rpt + off, zrows)])

    pltpu.sync_copy(dst_hbm.at[pl.ds(w * rows, rows)], dstbuf)
    plsc.subcore_barrier()

    def hist(j, _):
        pltpu.sync_copy(onesbuf, acc.at[dstbuf.at[j]], add=True)
        return 0

    lax.fori_loop(0, rows, hist, 0)
    plsc.subcore_barrier()
    pltpu.sync_copy(acc.at[pl.ds(s * rpt, rpt)],
                    deg_out.at[c, pl.ds(s * rpt, rpt), :])


def _deg_call(dstd, npad):
    mesh = plsc.VectorSubcoreMesh(core_axis_name="c", subcore_axis_name="s")
    rows = dstd.shape[0] // (_NC * _NS)
    fn = pl.kernel(
        _deg_body,
        out_type=jax.ShapeDtypeStruct((_NC, npad, 128), F32),
        mesh=mesh,
        scratch_types=[
            pltpu.VMEM((rows, _DEG_CHUNK), jnp.int32),
            pltpu.VMEM((_DEG_CHUNK, 128), F32),
            pltpu.VMEM((64, 128), F32),
            pltpu.VMEM_SHARED((npad, 128), F32),
        ],
    )
    return fn(dstd)


def _agg_body(yv_hbm, srcv_hbm, dstv_hbm, agg_out,
              srcbuf, dstbuf, rb0, rb1, acc, sem0, sem1):
    c = lax.axis_index("c")
    s = lax.axis_index("s")
    rows = srcbuf.shape[0]
    rpt = acc.shape[0] // _NS
    zrows = rb0.shape[0]

    def fill_zero(i, _):
        for k in range(128 // _L):
            rb0[i, pl.ds(k * _L, _L)] = jnp.zeros((_L,), F32)
        return 0

    lax.fori_loop(0, zrows, fill_zero, 0)
    offs = list(range(0, rpt - zrows + 1, zrows))
    if offs[-1] != rpt - zrows:
        offs.append(rpt - zrows)
    for off in offs:
        pltpu.sync_copy(rb0, acc.at[pl.ds(s * rpt + off, zrows)])
    plsc.subcore_barrier()

    def gather(j, rb, sem):
        pltpu.async_copy(yv_hbm.at[srcbuf.at[j]], rb, sem)

    def gwait(rb, sem):
        pltpu.make_async_copy(yv_hbm.at[srcbuf.at[0]], rb, sem).wait()

    def scat(j, rb):
        pltpu.sync_copy(rb, acc.at[dstbuf.at[j]], add=True)

    for h in range(2):
        base = (s * 2 + h) * rows
        pltpu.sync_copy(srcv_hbm.at[c, pl.ds(base, rows)], srcbuf)
        pltpu.sync_copy(dstv_hbm.at[pl.ds(base, rows)], dstbuf)

        def step(j, _):
            pltpu.async_copy(yv_hbm.at[srcbuf.at[j]], rb0, sem0).wait()
            scat(j, rb0)
            return 0

        lax.fori_loop(0, rows, step, 0)

    plsc.subcore_barrier()
    pltpu.sync_copy(acc.at[pl.ds(s * rpt, rpt)],
                    agg_out.at[c, pl.ds(s * rpt, rpt), :])


def _agg_call(yv, srcv, dstv, npad):
    mesh = plsc.VectorSubcoreMesh(core_axis_name="c", subcore_axis_name="s")
    rows = srcv.shape[1] // (_NS * 2)
    fn = pl.kernel(
        _agg_body,
        out_type=jax.ShapeDtypeStruct((_NC, npad, 128), F32),
        mesh=mesh,
        scratch_types=[
            pltpu.VMEM((rows, _AGG_CHUNK), jnp.int32),
            pltpu.VMEM((rows, _AGG_CHUNK), jnp.int32),
            pltpu.VMEM((_AGG_CHUNK, 128), F32),
            pltpu.VMEM((_AGG_CHUNK, 128), F32),
            pltpu.VMEM_SHARED((npad, 128), F32),
            pltpu.SemaphoreType.DMA,
            pltpu.SemaphoreType.DMA,
        ],
    )
    return fn(yv, srcv, dstv)


def _mm_scale_body(x_ref, w_ref, deg_ref, o_ref):
    deg = deg_ref[0][:, :1] + deg_ref[1][:, :1] + 1.0
    dinv = lax.rsqrt(deg)
    xw = jnp.dot(x_ref[...], w_ref[...], preferred_element_type=F32)
    o_ref[...] = xw * dinv


def _mm_scale(x, w, deg, blk):
    n, d_in = x.shape
    d_out = w.shape[1]
    grid = n // blk
    return pl.pallas_call(
        _mm_scale_body,
        grid=(grid,),
        in_specs=[
            pl.BlockSpec((blk, d_in), lambda i: (i, 0)),
            pl.BlockSpec((d_in, d_out), lambda i: (0, 0)),
            pl.BlockSpec((_NC, blk, 128), lambda i: (0, i, 0)),
        ],
        out_specs=pl.BlockSpec((blk, d_out), lambda i: (i, 0)),
        out_shape=jax.ShapeDtypeStruct((n, d_out), F32),
    )(x, w, deg)


def _fused_h(agg_ref, y_ref, deg_ref, b_ref, i, padcnt):
    blk = y_ref.shape[0]
    deg = deg_ref[0][:, :1] + deg_ref[1][:, :1] + 1.0
    dinv = lax.rsqrt(deg)
    aggf = jnp.concatenate([agg_ref[0], agg_ref[1]], axis=-1) + y_ref[...]
    if padcnt:
        row0 = (lax.broadcasted_iota(jnp.int32, (blk, 1), 0) == 0) & (i == 0)
        aggf = aggf - jnp.where(row0, float(padcnt), 0.0) * y_ref[...]
    return jnp.maximum(aggf * dinv + b_ref[...], 0.0), dinv


def _layer_body(agg_ref, y_ref, deg_ref, b_ref, w_ref, o_ref, *, padcnt):
    h, dinv = _fused_h(agg_ref, y_ref, deg_ref, b_ref, pl.program_id(0),
                       padcnt)
    o_ref[...] = jnp.dot(h, w_ref[...], preferred_element_type=F32) * dinv


def _layer(agg, y, deg, b, w, blk, padcnt):
    n, d = y.shape
    grid = n // blk
    return pl.pallas_call(
        functools.partial(_layer_body, padcnt=padcnt),
        grid=(grid,),
        in_specs=[
            pl.BlockSpec((_NC, blk, 128), lambda i: (0, i, 0)),
            pl.BlockSpec((blk, d), lambda i: (i, 0)),
            pl.BlockSpec((_NC, blk, 128), lambda i: (0, i, 0)),
            pl.BlockSpec((1, d), lambda i: (0, 0)),
            pl.BlockSpec((d, d), lambda i: (0, 0)),
        ],
        out_specs=pl.BlockSpec((blk, d), lambda i: (i, 0)),
        out_shape=jax.ShapeDtypeStruct((n, d), F32),
    )(agg, y, deg, b, w)


def _head_body(agg_ref, y_ref, deg_ref, b_ref, batch_ref, wfc_ref, bfc_ref,
               o_ref, pool_ref, *, nblk, g, padcnt):
    i = pl.program_id(0)
    h, _ = _fused_h(agg_ref, y_ref, deg_ref, b_ref, i, padcnt)
    bvec = batch_ref[0, 0, :]
    oh = (bvec[:, None] ==
          lax.broadcasted_iota(jnp.int32, (bvec.shape[0], g), 1)).astype(F32)
    seg = lax.dot_general(oh, h, (((0,), (0,)), ((), ())),
                          preferred_element_type=F32)

    @pl.when(i == 0)
    def _():
        pool_ref[...] = jnp.zeros_like(pool_ref)

    pool_ref[...] += seg

    @pl.when(i == nblk - 1)
    def _():
        o_ref[...] = (jnp.dot(pool_ref[...], wfc_ref[...],
                              preferred_element_type=F32) + bfc_ref[...])


def _head(agg, y, deg, b, batchv, wfc, bfc, blk, g, padcnt):
    n, d = y.shape
    d_out = wfc.shape[1]
    grid = n // blk
    body = functools.partial(_head_body, nblk=grid, g=g, padcnt=padcnt)
    return pl.pallas_call(
        body,
        grid=(grid,),
        in_specs=[
            pl.BlockSpec((_NC, blk, 128), lambda i: (0, i, 0)),
            pl.BlockSpec((blk, d), lambda i: (i, 0)),
            pl.BlockSpec((_NC, blk, 128), lambda i: (0, i, 0)),
            pl.BlockSpec((1, d), lambda i: (0, 0)),
            pl.BlockSpec((1, 1, blk), lambda i: (i, 0, 0)),
            pl.BlockSpec((d, d_out), lambda i: (0, 0)),
            pl.BlockSpec((1, d_out), lambda i: (0, 0)),
        ],
        out_specs=pl.BlockSpec((g, d_out), lambda i: (0, 0)),
        out_shape=jax.ShapeDtypeStruct((g, d_out), F32),
        scratch_shapes=[pltpu.VMEM((g, d), F32)],
    )(agg, y, deg, b, batchv, wfc, bfc)


def kernel(x, edge_index, batch, W1, b1, W2, b2, Wfc, bfc):
    n, d_in = x.shape
    e = edge_index.shape[1]
    d_h = W1.shape[1]
    g = 64
    blk = 1000
    assert d_h == 256 and n % blk == 0
    assert e % (_DEG_CHUNK * _NC * _NS * 8) == 0

    npad = _round_up(n, _NS * 8)
    e2 = _round_up(e, _AGG_CHUNK * _NS * 8)
    padcnt = e2 - e

    src = edge_index[0]
    dst = edge_index[1]
    pad = jnp.zeros((padcnt,), dst.dtype)
    dstd = dst.reshape(e // _DEG_CHUNK, _DEG_CHUNK)
    srcp = jnp.concatenate([src, pad]) * 2
    srcv = jnp.stack([srcp, srcp + 1]).reshape(
        2, e2 // _AGG_CHUNK, _AGG_CHUNK)
    dstv = jnp.concatenate([dst, pad]).reshape(e2 // _AGG_CHUNK, _AGG_CHUNK)
    batchv = batch.reshape(n // blk, 1, blk)

    deg = _deg_call(dstd, npad)
    y1 = _mm_scale(x, W1, deg, blk)
    agg1 = _agg_call(y1.reshape(2 * n, 128), srcv, dstv, npad)
    y2 = _layer(agg1, y1, deg, b1.reshape(1, -1), W2, blk, padcnt)
    agg2 = _agg_call(y2.reshape(2 * n, 128), srcv, dstv, npad)
    return _head(agg2, y2, deg, b2.reshape(1, -1), batchv, Wfc,
                 bfc.reshape(1, -1), blk, g, padcnt)


# speedup vs baseline: 7.4791x; 1.1451x over previous
import functools

import jax
import jax.numpy as jnp
from jax import lax
from jax.experimental import pallas as pl
from jax.experimental.pallas import tpu as pltpu
from jax.experimental.pallas import tpu_sc as plsc

F32 = jnp.float32

_NC = 2
_NS = 16
_L = 16

_DEG_CHUNK = 125
_AGG_CHUNK = 64


def _round_up(v, m):
    return (v + m - 1) // m * m


def _deg_body(dst_hbm, deg_out, dstbuf, onesbuf, zbuf, acc):
    c = lax.axis_index("c")
    s = lax.axis_index("s")
    w = c * _NS + s
    rows = dstbuf.shape[0]
    rpt = acc.shape[0] // _NS
    zrows = zbuf.shape[0]

    def fill_ones(i, _):
        for k in range(128 // _L):
            onesbuf[i, pl.ds(k * _L, _L)] = jnp.ones((_L,), F32)
        return 0

    lax.fori_loop(0, onesbuf.shape[0], fill_ones, 0)

    def fill_zero(i, _):
        for k in range(128 // _L):
            zbuf[i, pl.ds(k * _L, _L)] = jnp.zeros((_L,), F32)
        return 0

    lax.fori_loop(0, zrows, fill_zero, 0)
    for off in range(0, rpt - zrows + 1, zrows):
        pltpu.sync_copy(zbuf, acc.at[pl.ds(s * rpt + off, zrows)])

    pltpu.sync_copy(dst_hbm.at[pl.ds(w * rows, rows)], dstbuf)
    plsc.subcore_barrier()

    def hist(j, _):
        pltpu.sync_copy(onesbuf, acc.at[dstbuf.at[j]], add=True)
        return 0

    lax.fori_loop(0, rows, hist, 0)
    plsc.subcore_barrier()
    pltpu.sync_copy(acc.at[pl.ds(s * rpt, rpt)],
                    deg_out.at[c, pl.ds(s * rpt, rpt), :])


def _deg_call(dstd, npad):
    mesh = plsc.VectorSubcoreMesh(core_axis_name="c", subcore_axis_name="s")
    rows = dstd.shape[0] // (_NC * _NS)
    fn = pl.kernel(
        _deg_body,
        out_type=jax.ShapeDtypeStruct((_NC, npad, 128), F32),
        mesh=mesh,
        scratch_types=[
            pltpu.VMEM((rows, _DEG_CHUNK), jnp.int32),
            pltpu.VMEM((_DEG_CHUNK, 128), F32),
            pltpu.VMEM((64, 128), F32),
            pltpu.VMEM_SHARED((npad, 128), F32),
        ],
    )
    return fn(dstd)


def _agg_body(yv_hbm, srcv_hbm, dstv_hbm, agg_out,
              srcbuf, dstbuf, rb0, rb1, rb2, acc, sem0, sem1, sem2, sem3):
    c = lax.axis_index("c")
    s = lax.axis_index("s")
    rows = srcbuf.shape[0]
    rpt = acc.shape[0] // _NS
    zrows = rb0.shape[0]
    nstages = 4

    def fill_zero(i, _):
        for k in range(128 // _L):
            rb0[i, pl.ds(k * _L, _L)] = jnp.zeros((_L,), F32)
        return 0

    lax.fori_loop(0, zrows, fill_zero, 0)
    offs = list(range(0, rpt - zrows + 1, zrows))
    if offs[-1] != rpt - zrows:
        offs.append(rpt - zrows)
    for off in offs:
        pltpu.sync_copy(rb0, acc.at[pl.ds(s * rpt + off, zrows)])
    plsc.subcore_barrier()

    def gather(j, rb, sem):
        return pltpu.async_copy(yv_hbm.at[srcbuf.at[j]], rb, sem)

    def scat(j, rb):
        return pltpu.async_copy(rb, acc.at[dstbuf.at[j]], sem3, add=True)

    for st in range(nstages):
        base = (s * nstages + st) * rows
        pltpu.sync_copy(srcv_hbm.at[c, pl.ds(base, rows)], srcbuf)
        pltpu.sync_copy(dstv_hbm.at[pl.ds(base, rows)], dstbuf)

        def pipe(i, _):
            j0 = i * 3
            d0 = gather(j0, rb0, sem0)
            d1 = gather(j0 + 1, rb1, sem1)
            d2 = gather(j0 + 2, rb2, sem2)
            d0.wait()
            s0 = scat(j0, rb0)
            d1.wait()
            s1 = scat(j0 + 1, rb1)
            d2.wait()
            s2 = scat(j0 + 2, rb2)
            s0.wait()
            s1.wait()
            s2.wait()
            return 0

        lax.fori_loop(0, rows // 3, pipe, 0)
        for j in range(rows // 3 * 3, rows):
            d = gather(j, rb0, sem0)
            d.wait()
            scat(j, rb0).wait()

    plsc.subcore_barrier()
    pltpu.sync_copy(acc.at[pl.ds(s * rpt, rpt)],
                    agg_out.at[c, pl.ds(s * rpt, rpt), :])


def _agg_call(yv, srcv, dstv, npad):
    mesh = plsc.VectorSubcoreMesh(core_axis_name="c", subcore_axis_name="s")
    rows = srcv.shape[1] // (_NS * 4)
    fn = pl.kernel(
        _agg_body,
        out_type=jax.ShapeDtypeStruct((_NC, npad, 128), F32),
        mesh=mesh,
        scratch_types=[
            pltpu.VMEM((rows, _AGG_CHUNK), jnp.int32),
            pltpu.VMEM((rows, _AGG_CHUNK), jnp.int32),
            pltpu.VMEM((_AGG_CHUNK, 128), F32),
            pltpu.VMEM((_AGG_CHUNK, 128), F32),
            pltpu.VMEM((_AGG_CHUNK, 128), F32),
            pltpu.VMEM_SHARED((npad, 128), F32),
            pltpu.SemaphoreType.DMA,
            pltpu.SemaphoreType.DMA,
            pltpu.SemaphoreType.DMA,
            pltpu.SemaphoreType.DMA,
        ],
    )
    return fn(yv, srcv, dstv)


def _mm_scale_body(x_ref, w_ref, deg_ref, o_ref):
    deg = deg_ref[0][:, :1] + deg_ref[1][:, :1] + 1.0
    dinv = lax.rsqrt(deg)
    xw = jnp.dot(x_ref[...], w_ref[...], preferred_element_type=F32)
    o_ref[...] = xw * dinv


def _mm_scale(x, w, deg, blk):
    n, d_in = x.shape
    d_out = w.shape[1]
    grid = n // blk
    return pl.pallas_call(
        _mm_scale_body,
        grid=(grid,),
        in_specs=[
            pl.BlockSpec((blk, d_in), lambda i: (i, 0)),
            pl.BlockSpec((d_in, d_out), lambda i: (0, 0)),
            pl.BlockSpec((_NC, blk, 128), lambda i: (0, i, 0)),
        ],
        out_specs=pl.BlockSpec((blk, d_out), lambda i: (i, 0)),
        out_shape=jax.ShapeDtypeStruct((n, d_out), F32),
    )(x, w, deg)


def _fused_h(agg_ref, y_ref, deg_ref, b_ref, i, padcnt):
    blk = y_ref.shape[0]
    deg = deg_ref[0][:, :1] + deg_ref[1][:, :1] + 1.0
    dinv = lax.rsqrt(deg)
    aggf = jnp.concatenate([agg_ref[0], agg_ref[1]], axis=-1) + y_ref[...]
    if padcnt:
        row0 = (lax.broadcasted_iota(jnp.int32, (blk, 1), 0) == 0) & (i == 0)
        aggf = aggf - jnp.where(row0, float(padcnt), 0.0) * y_ref[...]
    return jnp.maximum(aggf * dinv + b_ref[...], 0.0), dinv


def _layer_body(agg_ref, y_ref, deg_ref, b_ref, w_ref, o_ref, *, padcnt):
    h, dinv = _fused_h(agg_ref, y_ref, deg_ref, b_ref, pl.program_id(0),
                       padcnt)
    o_ref[...] = jnp.dot(h, w_ref[...], preferred_element_type=F32) * dinv


def _layer(agg, y, deg, b, w, blk, padcnt):
    n, d = y.shape
    grid = n // blk
    return pl.pallas_call(
        functools.partial(_layer_body, padcnt=padcnt),
        grid=(grid,),
        in_specs=[
            pl.BlockSpec((_NC, blk, 128), lambda i: (0, i, 0)),
            pl.BlockSpec((blk, d), lambda i: (i, 0)),
            pl.BlockSpec((_NC, blk, 128), lambda i: (0, i, 0)),
            pl.BlockSpec((1, d), lambda i: (0, 0)),
            pl.BlockSpec((d, d), lambda i: (0, 0)),
        ],
        out_specs=pl.BlockSpec((blk, d), lambda i: (i, 0)),
        out_shape=jax.ShapeDtypeStruct((n, d), F32),
    )(agg, y, deg, b, w)


def _head_body(agg_ref, y_ref, deg_ref, b_ref, batch_ref, wfc_ref, bfc_ref,
               o_ref, pool_ref, *, nblk, g, padcnt):
    i = pl.program_id(0)
    h, _ = _fused_h(agg_ref, y_ref, deg_ref, b_ref, i, padcnt)
    bvec = batch_ref[0, 0, :]
    oh = (bvec[:, None] ==
          lax.broadcasted_iota(jnp.int32, (bvec.shape[0], g), 1)).astype(F32)
    seg = lax.dot_general(oh, h, (((0,), (0,)), ((), ())),
                          preferred_element_type=F32)

    @pl.when(i == 0)
    def _():
        pool_ref[...] = jnp.zeros_like(pool_ref)

    pool_ref[...] += seg

    @pl.when(i == nblk - 1)
    def _():
        o_ref[...] = (jnp.dot(pool_ref[...], wfc_ref[...],
                              preferred_element_type=F32) + bfc_ref[...])


def _head(agg, y, deg, b, batchv, wfc, bfc, blk, g, padcnt):
    n, d = y.shape
    d_out = wfc.shape[1]
    grid = n // blk
    body = functools.partial(_head_body, nblk=grid, g=g, padcnt=padcnt)
    return pl.pallas_call(
        body,
        grid=(grid,),
        in_specs=[
            pl.BlockSpec((_NC, blk, 128), lambda i: (0, i, 0)),
            pl.BlockSpec((blk, d), lambda i: (i, 0)),
            pl.BlockSpec((_NC, blk, 128), lambda i: (0, i, 0)),
            pl.BlockSpec((1, d), lambda i: (0, 0)),
            pl.BlockSpec((1, 1, blk), lambda i: (i, 0, 0)),
            pl.BlockSpec((d, d_out), lambda i: (0, 0)),
            pl.BlockSpec((1, d_out), lambda i: (0, 0)),
        ],
        out_specs=pl.BlockSpec((g, d_out), lambda i: (0, 0)),
        out_shape=jax.ShapeDtypeStruct((g, d_out), F32),
        scratch_shapes=[pltpu.VMEM((g, d), F32)],
    )(agg, y, deg, b, batchv, wfc, bfc)


def kernel(x, edge_index, batch, W1, b1, W2, b2, Wfc, bfc):
    n, d_in = x.shape
    e = edge_index.shape[1]
    d_h = W1.shape[1]
    g = 64
    blk = 1000
    assert d_h == 256 and n % blk == 0
    assert e % (_DEG_CHUNK * _NC * _NS * 8) == 0

    npad = _round_up(n, _NS * 8)
    e2 = _round_up(e, _AGG_CHUNK * _NS * 8)
    padcnt = e2 - e

    src = edge_index[0]
    dst = edge_index[1]
    pad = jnp.zeros((padcnt,), dst.dtype)
    dstd = dst.reshape(e // _DEG_CHUNK, _DEG_CHUNK)
    srcp = jnp.concatenate([src, pad]) * 2
    srcv = jnp.stack([srcp, srcp + 1]).reshape(
        2, e2 // _AGG_CHUNK, _AGG_CHUNK)
    dstv = jnp.concatenate([dst, pad]).reshape(e2 // _AGG_CHUNK, _AGG_CHUNK)
    batchv = batch.reshape(n // blk, 1, blk)

    deg = _deg_call(dstd, npad)
    y1 = _mm_scale(x, W1, deg, blk)
    agg1 = _agg_call(y1.reshape(2 * n, 128), srcv, dstv, npad)
    y2 = _layer(agg1, y1, deg, b1.reshape(1, -1), W2, blk, padcnt)
    agg2 = _agg_call(y2.reshape(2 * n, 128), srcv, dstv, npad)
    return _head(agg2, y2, deg, b2.reshape(1, -1), batchv, Wfc,
                 bfc.reshape(1, -1), blk, g, padcnt)


# speedup vs baseline: 8.4287x; 1.1270x over previous
import functools

import jax
import jax.numpy as jnp
from jax import lax
from jax.experimental import pallas as pl
from jax.experimental.pallas import tpu as pltpu
from jax.experimental.pallas import tpu_sc as plsc

F32 = jnp.float32

_NC = 2
_NS = 16
_L = 16

_DEG_CHUNK = 125
_AGG_CHUNK = 64


def _round_up(v, m):
    return (v + m - 1) // m * m


def _deg_body(dst_hbm, deg_out, dstbuf, onesbuf, zbuf, acc):
    c = lax.axis_index("c")
    s = lax.axis_index("s")
    w = c * _NS + s
    rows = dstbuf.shape[0]
    rpt = acc.shape[0] // _NS
    zrows = zbuf.shape[0]

    def fill_ones(i, _):
        for k in range(128 // _L):
            onesbuf[i, pl.ds(k * _L, _L)] = jnp.ones((_L,), F32)
        return 0

    lax.fori_loop(0, onesbuf.shape[0], fill_ones, 0)

    def fill_zero(i, _):
        for k in range(128 // _L):
            zbuf[i, pl.ds(k * _L, _L)] = jnp.zeros((_L,), F32)
        return 0

    lax.fori_loop(0, zrows, fill_zero, 0)
    for off in range(0, rpt - zrows + 1, zrows):
        pltpu.sync_copy(zbuf, acc.at[pl.ds(s * rpt + off, zrows)])

    pltpu.sync_copy(dst_hbm.at[pl.ds(w * rows, rows)], dstbuf)
    plsc.subcore_barrier()

    def hist(j, _):
        pltpu.sync_copy(onesbuf, acc.at[dstbuf.at[j]], add=True)
        return 0

    lax.fori_loop(0, rows, hist, 0)
    plsc.subcore_barrier()
    pltpu.sync_copy(acc.at[pl.ds(s * rpt, rpt)],
                    deg_out.at[c, pl.ds(s * rpt, rpt), :])


def _deg_call(dstd, npad):
    mesh = plsc.VectorSubcoreMesh(core_axis_name="c", subcore_axis_name="s")
    rows = dstd.shape[0] // (_NC * _NS)
    fn = pl.kernel(
        _deg_body,
        out_type=jax.ShapeDtypeStruct((_NC, npad, 128), F32),
        mesh=mesh,
        scratch_types=[
            pltpu.VMEM((rows, _DEG_CHUNK), jnp.int32),
            pltpu.VMEM((_DEG_CHUNK, 128), F32),
            pltpu.VMEM((64, 128), F32),
            pltpu.VMEM_SHARED((npad, 128), F32),
        ],
    )
    return fn(dstd)


def _agg_body(yv_hbm, srcv_hbm, dstv_hbm, agg_out,
              srcbuf, dstbuf, rb0, rb1, rb2, acc, sem0, sem1, sem2, sem3):
    c = lax.axis_index("c")
    s = lax.axis_index("s")
    rows = srcbuf.shape[0]
    rpt = acc.shape[0] // _NS
    zrows = rb0.shape[0]
    nstages = 4

    def fill_zero(i, _):
        for k in range(128 // _L):
            rb0[i, pl.ds(k * _L, _L)] = jnp.zeros((_L,), F32)
        return 0

    lax.fori_loop(0, zrows, fill_zero, 0)
    offs = list(range(0, rpt - zrows + 1, zrows))
    if offs[-1] != rpt - zrows:
        offs.append(rpt - zrows)
    for off in offs:
        pltpu.sync_copy(rb0, acc.at[pl.ds(s * rpt + off, zrows)])
    plsc.subcore_barrier()

    def gather(j, rb, sem):
        return pltpu.async_copy(yv_hbm.at[srcbuf.at[j]], rb, sem)

    def scat(j, rb):
        return pltpu.async_copy(rb, acc.at[dstbuf.at[j]], sem3, add=True)

    for st in range(nstages):
        base = (s * nstages + st) * rows
        pltpu.sync_copy(srcv_hbm.at[c, pl.ds(base, rows)], srcbuf)
        pltpu.sync_copy(dstv_hbm.at[pl.ds(base, rows)], dstbuf)

        rbs = (rb0, rb1, rb2)
        sems = (sem0, sem1, sem2)
        nu = rows // 2

        def window(j0):
            d = {0: gather(j0, rbs[0], sems[0]),
                 1: gather(j0 + 1, rbs[1], sems[1])}
            sv = {}
            for k in range(nu):
                d[k].wait()
                sv[k] = scat(j0 + k, rbs[k % 3])
                if k + 2 < nu:
                    if k >= 1:
                        sv[k - 1].wait()
                    d[k + 2] = gather(j0 + k + 2, rbs[(k + 2) % 3],
                                      sems[(k + 2) % 3])
            for k in range(max(0, nu - 3), nu):
                sv[k].wait()

        def pipe(i, _):
            window(i * nu)
            return 0

        lax.fori_loop(0, 2, pipe, 0)

    plsc.subcore_barrier()
    pltpu.sync_copy(acc.at[pl.ds(s * rpt, rpt)],
                    agg_out.at[c, pl.ds(s * rpt, rpt), :])


def _agg_call(yv, srcv, dstv, npad):
    mesh = plsc.VectorSubcoreMesh(core_axis_name="c", subcore_axis_name="s")
    rows = srcv.shape[1] // (_NS * 4)
    fn = pl.kernel(
        _agg_body,
        out_type=jax.ShapeDtypeStruct((_NC, npad, 128), F32),
        mesh=mesh,
        scratch_types=[
            pltpu.VMEM((rows, _AGG_CHUNK), jnp.int32),
            pltpu.VMEM((rows, _AGG_CHUNK), jnp.int32),
            pltpu.VMEM((_AGG_CHUNK, 128), F32),
            pltpu.VMEM((_AGG_CHUNK, 128), F32),
            pltpu.VMEM((_AGG_CHUNK, 128), F32),
            pltpu.VMEM_SHARED((npad, 128), F32),
            pltpu.SemaphoreType.DMA,
            pltpu.SemaphoreType.DMA,
            pltpu.SemaphoreType.DMA,
            pltpu.SemaphoreType.DMA,
        ],
    )
    return fn(yv, srcv, dstv)


def _mm_scale_body(x_ref, w_ref, deg_ref, o_ref):
    deg = deg_ref[0][:, :1] + deg_ref[1][:, :1] + 1.0
    dinv = lax.rsqrt(deg)
    xw = jnp.dot(x_ref[...], w_ref[...], preferred_element_type=F32)
    o_ref[...] = xw * dinv


def _mm_scale(x, w, deg, blk):
    n, d_in = x.shape
    d_out = w.shape[1]
    grid = n // blk
    return pl.pallas_call(
        _mm_scale_body,
        grid=(grid,),
        in_specs=[
            pl.BlockSpec((blk, d_in), lambda i: (i, 0)),
            pl.BlockSpec((d_in, d_out), lambda i: (0, 0)),
            pl.BlockSpec((_NC, blk, 128), lambda i: (0, i, 0)),
        ],
        out_specs=pl.BlockSpec((blk, d_out), lambda i: (i, 0)),
        out_shape=jax.ShapeDtypeStruct((n, d_out), F32),
    )(x, w, deg)


def _fused_h(agg_ref, y_ref, deg_ref, b_ref, i, padcnt):
    blk = y_ref.shape[0]
    deg = deg_ref[0][:, :1] + deg_ref[1][:, :1] + 1.0
    dinv = lax.rsqrt(deg)
    aggf = jnp.concatenate([agg_ref[0], agg_ref[1]], axis=-1) + y_ref[...]
    if padcnt:
        row0 = (lax.broadcasted_iota(jnp.int32, (blk, 1), 0) == 0) & (i == 0)
        aggf = aggf - jnp.where(row0, float(padcnt), 0.0) * y_ref[...]
    return jnp.maximum(aggf * dinv + b_ref[...], 0.0), dinv


def _layer_body(agg_ref, y_ref, deg_ref, b_ref, w_ref, o_ref, *, padcnt):
    h, dinv = _fused_h(agg_ref, y_ref, deg_ref, b_ref, pl.program_id(0),
                       padcnt)
    o_ref[...] = jnp.dot(h, w_ref[...], preferred_element_type=F32) * dinv


def _layer(agg, y, deg, b, w, blk, padcnt):
    n, d = y.shape
    grid = n // blk
    return pl.pallas_call(
        functools.partial(_layer_body, padcnt=padcnt),
        grid=(grid,),
        in_specs=[
            pl.BlockSpec((_NC, blk, 128), lambda i: (0, i, 0)),
            pl.BlockSpec((blk, d), lambda i: (i, 0)),
            pl.BlockSpec((_NC, blk, 128), lambda i: (0, i, 0)),
            pl.BlockSpec((1, d), lambda i: (0, 0)),
            pl.BlockSpec((d, d), lambda i: (0, 0)),
        ],
        out_specs=pl.BlockSpec((blk, d), lambda i: (i, 0)),
        out_shape=jax.ShapeDtypeStruct((n, d), F32),
    )(agg, y, deg, b, w)


def _head_body(agg_ref, y_ref, deg_ref, b_ref, batch_ref, wfc_ref, bfc_ref,
               o_ref, pool_ref, *, nblk, g, padcnt):
    i = pl.program_id(0)
    h, _ = _fused_h(agg_ref, y_ref, deg_ref, b_ref, i, padcnt)
    bvec = batch_ref[0, 0, :]
    oh = (bvec[:, None] ==
          lax.broadcasted_iota(jnp.int32, (bvec.shape[0], g), 1)).astype(F32)
    seg = lax.dot_general(oh, h, (((0,), (0,)), ((), ())),
                          preferred_element_type=F32)

    @pl.when(i == 0)
    def _():
        pool_ref[...] = jnp.zeros_like(pool_ref)

    pool_ref[...] += seg

    @pl.when(i == nblk - 1)
    def _():
        o_ref[...] = (jnp.dot(pool_ref[...], wfc_ref[...],
                              preferred_element_type=F32) + bfc_ref[...])


def _head(agg, y, deg, b, batchv, wfc, bfc, blk, g, padcnt):
    n, d = y.shape
    d_out = wfc.shape[1]
    grid = n // blk
    body = functools.partial(_head_body, nblk=grid, g=g, padcnt=padcnt)
    return pl.pallas_call(
        body,
        grid=(grid,),
        in_specs=[
            pl.BlockSpec((_NC, blk, 128), lambda i: (0, i, 0)),
            pl.BlockSpec((blk, d), lambda i: (i, 0)),
            pl.BlockSpec((_NC, blk, 128), lambda i: (0, i, 0)),
            pl.BlockSpec((1, d), lambda i: (0, 0)),
            pl.BlockSpec((1, 1, blk), lambda i: (i, 0, 0)),
            pl.BlockSpec((d, d_out), lambda i: (0, 0)),
            pl.BlockSpec((1, d_out), lambda i: (0, 0)),
        ],
        out_specs=pl.BlockSpec((g, d_out), lambda i: (0, 0)),
        out_shape=jax.ShapeDtypeStruct((g, d_out), F32),
        scratch_shapes=[pltpu.VMEM((g, d), F32)],
    )(agg, y, deg, b, batchv, wfc, bfc)


def kernel(x, edge_index, batch, W1, b1, W2, b2, Wfc, bfc):
    n, d_in = x.shape
    e = edge_index.shape[1]
    d_h = W1.shape[1]
    g = 64
    blk = 1000
    assert d_h == 256 and n % blk == 0
    assert e % (_DEG_CHUNK * _NC * _NS * 8) == 0

    npad = _round_up(n, _NS * 8)
    e2 = _round_up(e, _AGG_CHUNK * _NS * 8)
    padcnt = e2 - e

    src = edge_index[0]
    dst = edge_index[1]
    pad = jnp.zeros((padcnt,), dst.dtype)
    dstd = dst.reshape(e // _DEG_CHUNK, _DEG_CHUNK)
    srcp = jnp.concatenate([src, pad]) * 2
    srcv = jnp.stack([srcp, srcp + 1]).reshape(
        2, e2 // _AGG_CHUNK, _AGG_CHUNK)
    dstv = jnp.concatenate([dst, pad]).reshape(e2 // _AGG_CHUNK, _AGG_CHUNK)
    batchv = batch.reshape(n // blk, 1, blk)

    deg = _deg_call(dstd, npad)
    y1 = _mm_scale(x, W1, deg, blk)
    agg1 = _agg_call(y1.reshape(2 * n, 128), srcv, dstv, npad)
    y2 = _layer(agg1, y1, deg, b1.reshape(1, -1), W2, blk, padcnt)
    agg2 = _agg_call(y2.reshape(2 * n, 128), srcv, dstv, npad)
    return _head(agg2, y2, deg, b2.reshape(1, -1), batchv, Wfc,
                 bfc.reshape(1, -1), blk, g, padcnt)


# speedup vs baseline: 8.7022x; 1.0325x over previous
import functools

import jax
import jax.numpy as jnp
from jax import lax
from jax.experimental import pallas as pl
from jax.experimental.pallas import tpu as pltpu
from jax.experimental.pallas import tpu_sc as plsc

F32 = jnp.float32

_NC = 2
_NS = 16
_L = 16

_DEG_CHUNK = 125
_AGG_CHUNK = 64


def _round_up(v, m):
    return (v + m - 1) // m * m


def _deg_body(dst_hbm, deg_out, dstbuf, onesbuf, zbuf, acc):
    c = lax.axis_index("c")
    s = lax.axis_index("s")
    w = c * _NS + s
    rows = dstbuf.shape[0]
    rpt = acc.shape[0] // _NS
    zrows = zbuf.shape[0]

    def fill_ones(i, _):
        for k in range(128 // _L):
            onesbuf[i, pl.ds(k * _L, _L)] = jnp.ones((_L,), F32)
        return 0

    lax.fori_loop(0, onesbuf.shape[0], fill_ones, 0)

    def fill_zero(i, _):
        for k in range(128 // _L):
            zbuf[i, pl.ds(k * _L, _L)] = jnp.zeros((_L,), F32)
        return 0

    lax.fori_loop(0, zrows, fill_zero, 0)
    for off in range(0, rpt - zrows + 1, zrows):
        pltpu.sync_copy(zbuf, acc.at[pl.ds(s * rpt + off, zrows)])

    pltpu.sync_copy(dst_hbm.at[pl.ds(w * rows, rows)], dstbuf)
    plsc.subcore_barrier()

    def hist(j, _):
        pltpu.sync_copy(onesbuf, acc.at[dstbuf.at[j]], add=True)
        return 0

    lax.fori_loop(0, rows, hist, 0)
    plsc.subcore_barrier()
    pltpu.sync_copy(acc.at[pl.ds(s * rpt, rpt)],
                    deg_out.at[c, pl.ds(s * rpt, rpt), :])


def _deg_call(dstd, npad):
    mesh = plsc.VectorSubcoreMesh(core_axis_name="c", subcore_axis_name="s")
    rows = dstd.shape[0] // (_NC * _NS)
    fn = pl.kernel(
        _deg_body,
        out_type=jax.ShapeDtypeStruct((_NC, npad, 128), F32),
        mesh=mesh,
        scratch_types=[
            pltpu.VMEM((rows, _DEG_CHUNK), jnp.int32),
            pltpu.VMEM((_DEG_CHUNK, 128), F32),
            pltpu.VMEM((64, 128), F32),
            pltpu.VMEM_SHARED((npad, 128), F32),
        ],
    )
    return fn(dstd)


def _agg_body(yv_hbm, srcv_hbm, dstv_hbm, agg_out,
              srcbuf, dstbuf, rb0, rb1, rb2, acc, sem0, sem1, sem2, sem3):
    c = lax.axis_index("c")
    s = lax.axis_index("s")
    rows = srcbuf.shape[0]
    rpt = acc.shape[0] // _NS
    zrows = rb0.shape[0]
    nstages = 4

    def fill_zero(i, _):
        for k in range(128 // _L):
            rb0[i, pl.ds(k * _L, _L)] = jnp.zeros((_L,), F32)
        return 0

    lax.fori_loop(0, zrows, fill_zero, 0)
    offs = list(range(0, rpt - zrows + 1, zrows))
    if offs[-1] != rpt - zrows:
        offs.append(rpt - zrows)
    for off in offs:
        pltpu.sync_copy(rb0, acc.at[pl.ds(s * rpt + off, zrows)])
    plsc.subcore_barrier()

    def gather(j, rb, sem):
        return pltpu.async_copy(yv_hbm.at[srcbuf.at[j]], rb, sem)

    def scat(j, rb):
        return pltpu.async_copy(rb, acc.at[dstbuf.at[j]], sem3, add=True)

    for st in range(nstages):
        base = (s * nstages + st) * rows
        pltpu.sync_copy(srcv_hbm.at[c, pl.ds(base, rows)], srcbuf)
        pltpu.sync_copy(dstv_hbm.at[pl.ds(base, rows)], dstbuf)

        rbs = (rb0, rb1, rb2)
        sems = (sem0, sem1, sem2)
        nu = rows // 2

        def window(j0):
            d = {0: gather(j0, rbs[0], sems[0]),
                 1: gather(j0 + 1, rbs[1], sems[1])}
            sv = {}
            for k in range(nu):
                d[k].wait()
                sv[k] = scat(j0 + k, rbs[k % 3])
                if k + 2 < nu:
                    if k >= 1:
                        sv[k - 1].wait()
                    d[k + 2] = gather(j0 + k + 2, rbs[(k + 2) % 3],
                                      sems[(k + 2) % 3])
            for k in range(max(0, nu - 3), nu):
                sv[k].wait()

        def pipe(i, _):
            window(i * nu)
            return 0

        lax.fori_loop(0, 2, pipe, 0)

    plsc.subcore_barrier()
    pltpu.sync_copy(acc.at[pl.ds(s * rpt, rpt)],
                    agg_out.at[c, pl.ds(s * rpt, rpt), :])


def _agg_call(yv, srcv, dstv, npad):
    mesh = plsc.VectorSubcoreMesh(core_axis_name="c", subcore_axis_name="s")
    rows = srcv.shape[1] // (_NS * 4)
    fn = pl.kernel(
        _agg_body,
        out_type=jax.ShapeDtypeStruct((_NC, npad, 128), F32),
        mesh=mesh,
        scratch_types=[
            pltpu.VMEM((rows, _AGG_CHUNK), jnp.int32),
            pltpu.VMEM((rows, _AGG_CHUNK), jnp.int32),
            pltpu.VMEM((_AGG_CHUNK, 128), F32),
            pltpu.VMEM((_AGG_CHUNK, 128), F32),
            pltpu.VMEM((_AGG_CHUNK, 128), F32),
            pltpu.VMEM_SHARED((npad, 128), F32),
            pltpu.SemaphoreType.DMA,
            pltpu.SemaphoreType.DMA,
            pltpu.SemaphoreType.DMA,
            pltpu.SemaphoreType.DMA,
        ],
    )
    return fn(yv, srcv, dstv)


def _mm_scale_body(x_ref, w_ref, deg_ref, o_ref):
    deg = deg_ref[0][:, :1] + deg_ref[1][:, :1] + 1.0
    dinv = lax.rsqrt(deg)
    xw = jnp.dot(x_ref[...], w_ref[...], preferred_element_type=F32) * dinv
    o_ref[0, ...] = xw[:, :128]
    o_ref[1, ...] = xw[:, 128:]


def _mm_scale(x, w, deg, blk):
    n, d_in = x.shape
    d_out = w.shape[1]
    grid = n // blk
    return pl.pallas_call(
        _mm_scale_body,
        grid=(grid,),
        in_specs=[
            pl.BlockSpec((blk, d_in), lambda i: (i, 0)),
            pl.BlockSpec((d_in, d_out), lambda i: (0, 0)),
            pl.BlockSpec((_NC, blk, 128), lambda i: (0, i, 0)),
        ],
        out_specs=pl.BlockSpec((2, blk, 128), lambda i: (0, i, 0)),
        out_shape=jax.ShapeDtypeStruct((2, n, 128), F32),
    )(x, w, deg)


def _fused_h(agg_ref, y_ref, deg_ref, b_ref, i, padcnt):
    blk = y_ref.shape[1]
    deg = deg_ref[0][:, :1] + deg_ref[1][:, :1] + 1.0
    dinv = lax.rsqrt(deg)
    yf = jnp.concatenate([y_ref[0], y_ref[1]], axis=-1)
    aggf = jnp.concatenate([agg_ref[0], agg_ref[1]], axis=-1) + yf
    if padcnt:
        row0 = (lax.broadcasted_iota(jnp.int32, (blk, 1), 0) == 0) & (i == 0)
        aggf = aggf - jnp.where(row0, float(padcnt), 0.0) * yf
    return jnp.maximum(aggf * dinv + b_ref[...], 0.0), dinv


def _layer_body(agg_ref, y_ref, deg_ref, b_ref, w_ref, o_ref, *, padcnt):
    h, dinv = _fused_h(agg_ref, y_ref, deg_ref, b_ref, pl.program_id(0),
                       padcnt)
    y2 = jnp.dot(h, w_ref[...], preferred_element_type=F32) * dinv
    o_ref[0, ...] = y2[:, :128]
    o_ref[1, ...] = y2[:, 128:]


def _layer(agg, y, deg, b, w, blk, padcnt):
    n = y.shape[1]
    d = w.shape[0]
    grid = n // blk
    return pl.pallas_call(
        functools.partial(_layer_body, padcnt=padcnt),
        grid=(grid,),
        in_specs=[
            pl.BlockSpec((_NC, blk, 128), lambda i: (0, i, 0)),
            pl.BlockSpec((2, blk, 128), lambda i: (0, i, 0)),
            pl.BlockSpec((_NC, blk, 128), lambda i: (0, i, 0)),
            pl.BlockSpec((1, d), lambda i: (0, 0)),
            pl.BlockSpec((d, d), lambda i: (0, 0)),
        ],
        out_specs=pl.BlockSpec((2, blk, 128), lambda i: (0, i, 0)),
        out_shape=jax.ShapeDtypeStruct((2, n, 128), F32),
    )(agg, y, deg, b, w)


def _head_body(agg_ref, y_ref, deg_ref, b_ref, batch_ref, wfc_ref, bfc_ref,
               o_ref, pool_ref, *, nblk, g, padcnt):
    i = pl.program_id(0)
    h, _ = _fused_h(agg_ref, y_ref, deg_ref, b_ref, i, padcnt)
    bvec = batch_ref[0, 0, :]
    oh = (bvec[:, None] ==
          lax.broadcasted_iota(jnp.int32, (bvec.shape[0], g), 1)).astype(F32)
    seg = lax.dot_general(oh, h, (((0,), (0,)), ((), ())),
                          preferred_element_type=F32)

    @pl.when(i == 0)
    def _():
        pool_ref[...] = jnp.zeros_like(pool_ref)

    pool_ref[...] += seg

    @pl.when(i == nblk - 1)
    def _():
        o_ref[...] = (jnp.dot(pool_ref[...], wfc_ref[...],
                              preferred_element_type=F32) + bfc_ref[...])


def _head(agg, y, deg, b, batchv, wfc, bfc, blk, g, padcnt):
    n = y.shape[1]
    d = wfc.shape[0]
    d_out = wfc.shape[1]
    grid = n // blk
    body = functools.partial(_head_body, nblk=grid, g=g, padcnt=padcnt)
    return pl.pallas_call(
        body,
        grid=(grid,),
        in_specs=[
            pl.BlockSpec((_NC, blk, 128), lambda i: (0, i, 0)),
            pl.BlockSpec((2, blk, 128), lambda i: (0, i, 0)),
            pl.BlockSpec((_NC, blk, 128), lambda i: (0, i, 0)),
            pl.BlockSpec((1, d), lambda i: (0, 0)),
            pl.BlockSpec((1, 1, blk), lambda i: (i, 0, 0)),
            pl.BlockSpec((d, d_out), lambda i: (0, 0)),
            pl.BlockSpec((1, d_out), lambda i: (0, 0)),
        ],
        out_specs=pl.BlockSpec((g, d_out), lambda i: (0, 0)),
        out_shape=jax.ShapeDtypeStruct((g, d_out), F32),
        scratch_shapes=[pltpu.VMEM((g, d), F32)],
    )(agg, y, deg, b, batchv, wfc, bfc)


def kernel(x, edge_index, batch, W1, b1, W2, b2, Wfc, bfc):
    n, d_in = x.shape
    e = edge_index.shape[1]
    d_h = W1.shape[1]
    g = 64
    blk = 1000
    assert d_h == 256 and n % blk == 0
    assert e % (_DEG_CHUNK * _NC * _NS * 8) == 0

    npad = _round_up(n, _NS * 8)
    e2 = _round_up(e, _AGG_CHUNK * _NS * 8)
    padcnt = e2 - e

    src = edge_index[0]
    dst = edge_index[1]
    pad = jnp.zeros((padcnt,), dst.dtype)
    dstd = dst.reshape(e // _DEG_CHUNK, _DEG_CHUNK)
    srcp = jnp.concatenate([src, pad])
    srcv = jnp.stack([srcp, srcp + n]).reshape(
        2, e2 // _AGG_CHUNK, _AGG_CHUNK)
    dstv = jnp.concatenate([dst, pad]).reshape(e2 // _AGG_CHUNK, _AGG_CHUNK)
    batchv = batch.reshape(n // blk, 1, blk)

    deg = _deg_call(dstd, npad)
    y1 = _mm_scale(x, W1, deg, blk)
    agg1 = _agg_call(y1.reshape(2 * n, 128), srcv, dstv, npad)
    y2 = _layer(agg1, y1, deg, b1.reshape(1, -1), W2, blk, padcnt)
    agg2 = _agg_call(y2.reshape(2 * n, 128), srcv, dstv, npad)
    return _head(agg2, y2, deg, b2.reshape(1, -1), batchv, Wfc,
                 bfc.reshape(1, -1), blk, g, padcnt)


# speedup vs baseline: 8.7138x; 1.0013x over previous
import functools

import jax
import jax.numpy as jnp
from jax import lax
from jax.experimental import pallas as pl
from jax.experimental.pallas import tpu as pltpu
from jax.experimental.pallas import tpu_sc as plsc

F32 = jnp.float32

_NC = 2
_NS = 16
_L = 16

_DEG_CHUNK = 125
_AGG_CHUNK = 64


def _round_up(v, m):
    return (v + m - 1) // m * m


def _deg_body(dst_hbm, deg_out, dstbuf, onesbuf, zbuf, acc, semh):
    c = lax.axis_index("c")
    s = lax.axis_index("s")
    w = c * _NS + s
    rows = dstbuf.shape[0]
    rpt = acc.shape[0] // _NS
    zrows = zbuf.shape[0]

    def fill_ones(i, _):
        for k in range(128 // _L):
            onesbuf[i, pl.ds(k * _L, _L)] = jnp.ones((_L,), F32)
        return 0

    lax.fori_loop(0, onesbuf.shape[0], fill_ones, 0)

    def fill_zero(i, _):
        for k in range(128 // _L):
            zbuf[i, pl.ds(k * _L, _L)] = jnp.zeros((_L,), F32)
        return 0

    lax.fori_loop(0, zrows, fill_zero, 0)
    for off in range(0, rpt - zrows + 1, zrows):
        pltpu.sync_copy(zbuf, acc.at[pl.ds(s * rpt + off, zrows)])

    pltpu.sync_copy(dst_hbm.at[pl.ds(w * rows, rows)], dstbuf)
    plsc.subcore_barrier()

    hd = [pltpu.async_copy(onesbuf, acc.at[dstbuf.at[j]], semh, add=True)
          for j in range(rows)]
    for d in hd:
        d.wait()
    plsc.subcore_barrier()
    pltpu.sync_copy(acc.at[pl.ds(s * rpt, rpt)],
                    deg_out.at[c, pl.ds(s * rpt, rpt), :])


def _deg_call(dstd, npad):
    mesh = plsc.VectorSubcoreMesh(core_axis_name="c", subcore_axis_name="s")
    rows = dstd.shape[0] // (_NC * _NS)
    fn = pl.kernel(
        _deg_body,
        out_type=jax.ShapeDtypeStruct((_NC, npad, 128), F32),
        mesh=mesh,
        scratch_types=[
            pltpu.VMEM((rows, _DEG_CHUNK), jnp.int32),
            pltpu.VMEM((_DEG_CHUNK, 128), F32),
            pltpu.VMEM((64, 128), F32),
            pltpu.VMEM_SHARED((npad, 128), F32),
            pltpu.SemaphoreType.DMA,
        ],
    )
    return fn(dstd)


def _agg_body(yv_hbm, srcv_hbm, dstv_hbm, agg_out,
              srcbuf, dstbuf, rb0, rb1, rb2, acc, sem0, sem1, sem2, sem3):
    c = lax.axis_index("c")
    s = lax.axis_index("s")
    rows = srcbuf.shape[0]
    rpt = acc.shape[0] // _NS
    zrows = rb0.shape[0]
    nstages = 4

    def fill_zero(i, _):
        for k in range(128 // _L):
            rb0[i, pl.ds(k * _L, _L)] = jnp.zeros((_L,), F32)
        return 0

    lax.fori_loop(0, zrows, fill_zero, 0)
    offs = list(range(0, rpt - zrows + 1, zrows))
    if offs[-1] != rpt - zrows:
        offs.append(rpt - zrows)
    zd = [pltpu.async_copy(rb0, acc.at[pl.ds(s * rpt + off, zrows)], sem3)
          for off in offs]
    for d in zd:
        d.wait()
    plsc.subcore_barrier()

    def gather(j, rb, sem):
        return pltpu.async_copy(yv_hbm.at[srcbuf.at[j]], rb, sem)

    def scat(j, rb):
        return pltpu.async_copy(rb, acc.at[dstbuf.at[j]], sem3, add=True)

    for st in range(nstages):
        base = (s * nstages + st) * rows
        pltpu.sync_copy(srcv_hbm.at[c, pl.ds(base, rows)], srcbuf)
        pltpu.sync_copy(dstv_hbm.at[pl.ds(base, rows)], dstbuf)

        rbs = (rb0, rb1, rb2)
        sems = (sem0, sem1, sem2)
        nu = rows // 2

        def window(j0):
            d = {0: gather(j0, rbs[0], sems[0]),
                 1: gather(j0 + 1, rbs[1], sems[1])}
            sv = {}
            for k in range(nu):
                d[k].wait()
                sv[k] = scat(j0 + k, rbs[k % 3])
                if k + 2 < nu:
                    if k >= 1:
                        sv[k - 1].wait()
                    d[k + 2] = gather(j0 + k + 2, rbs[(k + 2) % 3],
                                      sems[(k + 2) % 3])
            for k in range(max(0, nu - 3), nu):
                sv[k].wait()

        def pipe(i, _):
            window(i * nu)
            return 0

        lax.fori_loop(0, 2, pipe, 0)

    plsc.subcore_barrier()
    pltpu.sync_copy(acc.at[pl.ds(s * rpt, rpt)],
                    agg_out.at[c, pl.ds(s * rpt, rpt), :])


def _agg_call(yv, srcv, dstv, npad):
    mesh = plsc.VectorSubcoreMesh(core_axis_name="c", subcore_axis_name="s")
    rows = srcv.shape[1] // (_NS * 4)
    fn = pl.kernel(
        _agg_body,
        out_type=jax.ShapeDtypeStruct((_NC, npad, 128), F32),
        mesh=mesh,
        scratch_types=[
            pltpu.VMEM((rows, _AGG_CHUNK), jnp.int32),
            pltpu.VMEM((rows, _AGG_CHUNK), jnp.int32),
            pltpu.VMEM((_AGG_CHUNK, 128), F32),
            pltpu.VMEM((_AGG_CHUNK, 128), F32),
            pltpu.VMEM((_AGG_CHUNK, 128), F32),
            pltpu.VMEM_SHARED((npad, 128), F32),
            pltpu.SemaphoreType.DMA,
            pltpu.SemaphoreType.DMA,
            pltpu.SemaphoreType.DMA,
            pltpu.SemaphoreType.DMA,
        ],
    )
    return fn(yv, srcv, dstv)


def _mm_scale_body(x_ref, w_ref, deg_ref, o_ref):
    deg = deg_ref[0][:, :1] + deg_ref[1][:, :1] + 1.0
    dinv = lax.rsqrt(deg)
    xw = jnp.dot(x_ref[...], w_ref[...], preferred_element_type=F32) * dinv
    o_ref[0, ...] = xw[:, :128]
    o_ref[1, ...] = xw[:, 128:]


def _mm_scale(x, w, deg, blk):
    n, d_in = x.shape
    d_out = w.shape[1]
    grid = n // blk
    return pl.pallas_call(
        _mm_scale_body,
        grid=(grid,),
        in_specs=[
            pl.BlockSpec((blk, d_in), lambda i: (i, 0)),
            pl.BlockSpec((d_in, d_out), lambda i: (0, 0)),
            pl.BlockSpec((_NC, blk, 128), lambda i: (0, i, 0)),
        ],
        out_specs=pl.BlockSpec((2, blk, 128), lambda i: (0, i, 0)),
        out_shape=jax.ShapeDtypeStruct((2, n, 128), F32),
    )(x, w, deg)


def _fused_h(agg_ref, y_ref, deg_ref, b_ref, i, padcnt):
    blk = y_ref.shape[1]
    deg = deg_ref[0][:, :1] + deg_ref[1][:, :1] + 1.0
    dinv = lax.rsqrt(deg)
    yf = jnp.concatenate([y_ref[0], y_ref[1]], axis=-1)
    aggf = jnp.concatenate([agg_ref[0], agg_ref[1]], axis=-1) + yf
    if padcnt:
        row0 = (lax.broadcasted_iota(jnp.int32, (blk, 1), 0) == 0) & (i == 0)
        aggf = aggf - jnp.where(row0, float(padcnt), 0.0) * yf
    return jnp.maximum(aggf * dinv + b_ref[...], 0.0), dinv


def _layer_body(agg_ref, y_ref, deg_ref, b_ref, w_ref, o_ref, *, padcnt):
    h, dinv = _fused_h(agg_ref, y_ref, deg_ref, b_ref, pl.program_id(0),
                       padcnt)
    y2 = jnp.dot(h, w_ref[...], preferred_element_type=F32) * dinv
    o_ref[0, ...] = y2[:, :128]
    o_ref[1, ...] = y2[:, 128:]


def _layer(agg, y, deg, b, w, blk, padcnt):
    n = y.shape[1]
    d = w.shape[0]
    grid = n // blk
    return pl.pallas_call(
        functools.partial(_layer_body, padcnt=padcnt),
        grid=(grid,),
        in_specs=[
            pl.BlockSpec((_NC, blk, 128), lambda i: (0, i, 0)),
            pl.BlockSpec((2, blk, 128), lambda i: (0, i, 0)),
            pl.BlockSpec((_NC, blk, 128), lambda i: (0, i, 0)),
            pl.BlockSpec((1, d), lambda i: (0, 0)),
            pl.BlockSpec((d, d), lambda i: (0, 0)),
        ],
        out_specs=pl.BlockSpec((2, blk, 128), lambda i: (0, i, 0)),
        out_shape=jax.ShapeDtypeStruct((2, n, 128), F32),
    )(agg, y, deg, b, w)


def _head_body(agg_ref, y_ref, deg_ref, b_ref, batch_ref, wfc_ref, bfc_ref,
               o_ref, pool_ref, *, nblk, g, padcnt):
    i = pl.program_id(0)
    h, _ = _fused_h(agg_ref, y_ref, deg_ref, b_ref, i, padcnt)
    bvec = batch_ref[0, 0, :]
    oh = (bvec[:, None] ==
          lax.broadcasted_iota(jnp.int32, (bvec.shape[0], g), 1)).astype(F32)
    seg = lax.dot_general(oh, h, (((0,), (0,)), ((), ())),
                          preferred_element_type=F32)

    @pl.when(i == 0)
    def _():
        pool_ref[...] = jnp.zeros_like(pool_ref)

    pool_ref[...] += seg

    @pl.when(i == nblk - 1)
    def _():
        o_ref[...] = (jnp.dot(pool_ref[...], wfc_ref[...],
                              preferred_element_type=F32) + bfc_ref[...])


def _head(agg, y, deg, b, batchv, wfc, bfc, blk, g, padcnt):
    n = y.shape[1]
    d = wfc.shape[0]
    d_out = wfc.shape[1]
    grid = n // blk
    body = functools.partial(_head_body, nblk=grid, g=g, padcnt=padcnt)
    return pl.pallas_call(
        body,
        grid=(grid,),
        in_specs=[
            pl.BlockSpec((_NC, blk, 128), lambda i: (0, i, 0)),
            pl.BlockSpec((2, blk, 128), lambda i: (0, i, 0)),
            pl.BlockSpec((_NC, blk, 128), lambda i: (0, i, 0)),
            pl.BlockSpec((1, d), lambda i: (0, 0)),
            pl.BlockSpec((1, 1, blk), lambda i: (i, 0, 0)),
            pl.BlockSpec((d, d_out), lambda i: (0, 0)),
            pl.BlockSpec((1, d_out), lambda i: (0, 0)),
        ],
        out_specs=pl.BlockSpec((g, d_out), lambda i: (0, 0)),
        out_shape=jax.ShapeDtypeStruct((g, d_out), F32),
        scratch_shapes=[pltpu.VMEM((g, d), F32)],
    )(agg, y, deg, b, batchv, wfc, bfc)


def kernel(x, edge_index, batch, W1, b1, W2, b2, Wfc, bfc):
    n, d_in = x.shape
    e = edge_index.shape[1]
    d_h = W1.shape[1]
    g = 64
    blk = 1000
    assert d_h == 256 and n % blk == 0
    assert e % (_DEG_CHUNK * _NC * _NS * 8) == 0

    npad = _round_up(n, _NS * 8)
    e2 = _round_up(e, _AGG_CHUNK * _NS * 8)
    padcnt = e2 - e

    src = edge_index[0]
    dst = edge_index[1]
    pad = jnp.zeros((padcnt,), dst.dtype)
    dstd = dst.reshape(e // _DEG_CHUNK, _DEG_CHUNK)
    srcp = jnp.concatenate([src, pad])
    srcv = jnp.stack([srcp, srcp + n]).reshape(
        2, e2 // _AGG_CHUNK, _AGG_CHUNK)
    dstv = jnp.concatenate([dst, pad]).reshape(e2 // _AGG_CHUNK, _AGG_CHUNK)
    batchv = batch.reshape(n // blk, 1, blk)

    deg = _deg_call(dstd, npad)
    y1 = _mm_scale(x, W1, deg, blk)
    agg1 = _agg_call(y1.reshape(2 * n, 128), srcv, dstv, npad)
    y2 = _layer(agg1, y1, deg, b1.reshape(1, -1), W2, blk, padcnt)
    agg2 = _agg_call(y2.reshape(2 * n, 128), srcv, dstv, npad)
    return _head(agg2, y2, deg, b2.reshape(1, -1), batchv, Wfc,
                 bfc.reshape(1, -1), blk, g, padcnt)
